# baseline (speedup 1.0000x reference)
import jax
import jax.numpy as jnp
from jax.experimental import pallas as pl


def _pos_body(col_ref, row_ref, out_ref):
    w, d = col_ref.shape
    h = row_ref.shape[0]
    col = col_ref[...]
    row = row_ref[...]
    left = jnp.broadcast_to(col[None, :, :], (h, w, d)).reshape(h * w, d)
    right = jnp.broadcast_to(row[:, None, :], (h, w, d)).reshape(h * w, d)
    out_ref[0] = jnp.concatenate([left, right], axis=-1)


def kernel(x, row_embed, col_embed):
    b = x.shape[0]
    h, w = x.shape[-3], x.shape[-2]
    d = row_embed.shape[1]
    col = col_embed[:w]
    row = row_embed[:h]
    return pl.pallas_call(
        _pos_body,
        grid=(b,),
        in_specs=[
            pl.BlockSpec((w, d), lambda i: (0, 0)),
            pl.BlockSpec((h, d), lambda i: (0, 0)),
        ],
        out_specs=pl.BlockSpec((1, h * w, 2 * d), lambda i: (i, 0, 0)),
        out_shape=jax.ShapeDtypeStruct((b, h * w, 2 * d), jnp.float32),
    )(col, row)


# speedup vs baseline: 1.0052x; 1.0052x over previous
import jax
import jax.numpy as jnp
from jax.experimental import pallas as pl


def _pos_body(col_ref, row_ref, out_ref):
    w, d = col_ref.shape
    h = row_ref.shape[0]
    col = col_ref[...]
    row = row_ref[...]
    left = jnp.broadcast_to(col[None, :, :], (h, w, d)).reshape(h * w, d)
    right = jnp.broadcast_to(row[:, None, :], (h, w, d)).reshape(h * w, d)
    out_ref[0, :, 0:d] = left
    out_ref[0, :, d:2 * d] = right


def kernel(x, row_embed, col_embed):
    b = x.shape[0]
    h, w = x.shape[-3], x.shape[-2]
    d = row_embed.shape[1]
    col = col_embed[:w]
    row = row_embed[:h]
    return pl.pallas_call(
        _pos_body,
        grid=(b,),
        in_specs=[
            pl.BlockSpec((w, d), lambda i: (0, 0)),
            pl.BlockSpec((h, d), lambda i: (0, 0)),
        ],
        out_specs=pl.BlockSpec((1, h * w, 2 * d), lambda i: (i, 0, 0)),
        out_shape=jax.ShapeDtypeStruct((b, h * w, 2 * d), jnp.float32),
    )(col, row)


# speedup vs baseline: 1.0094x; 1.0041x over previous
import jax
import jax.numpy as jnp
from jax.experimental import pallas as pl


_BB = 4


def _pos_body(col_ref, row_ref, out_ref):
    w, d = col_ref.shape
    h = row_ref.shape[0]
    col = col_ref[...]
    row = row_ref[...]
    left = jnp.broadcast_to(col[None, :, :], (h, w, d)).reshape(h * w, d)
    right = jnp.broadcast_to(row[:, None, :], (h, w, d)).reshape(h * w, d)
    for bb in range(_BB):
        out_ref[bb, :, 0:d] = left
        out_ref[bb, :, d:2 * d] = right


def kernel(x, row_embed, col_embed):
    b = x.shape[0]
    h, w = x.shape[-3], x.shape[-2]
    d = row_embed.shape[1]
    col = col_embed[:w]
    row = row_embed[:h]
    return pl.pallas_call(
        _pos_body,
        grid=(b // _BB,),
        in_specs=[
            pl.BlockSpec((w, d), lambda i: (0, 0)),
            pl.BlockSpec((h, d), lambda i: (0, 0)),
        ],
        out_specs=pl.BlockSpec((_BB, h * w, 2 * d), lambda i: (i, 0, 0)),
        out_shape=jax.ShapeDtypeStruct((b, h * w, 2 * d), jnp.float32),
    )(col, row)


# speedup vs baseline: 1.1221x; 1.1117x over previous
import jax
import jax.numpy as jnp
from jax.experimental import pallas as pl
from jax.experimental.pallas import tpu as pltpu


def _pos_body(col_ref, row_ref, out_ref, scratch, sem):
    w, d = col_ref.shape
    h = row_ref.shape[0]
    b = out_ref.shape[0]
    col = col_ref[...]
    row = row_ref[...]
    left = jnp.broadcast_to(col[None, :, :], (h, w, d)).reshape(h * w, d)
    right = jnp.broadcast_to(row[:, None, :], (h, w, d)).reshape(h * w, d)
    scratch[:, 0:d] = left
    scratch[:, d:2 * d] = right
    copies = [
        pltpu.make_async_copy(scratch, out_ref.at[i], sem.at[i])
        for i in range(b)
    ]
    for c in copies:
        c.start()
    for c in copies:
        c.wait()


def kernel(x, row_embed, col_embed):
    b = x.shape[0]
    h, w = x.shape[-3], x.shape[-2]
    d = row_embed.shape[1]
    col = col_embed[:w]
    row = row_embed[:h]
    return pl.pallas_call(
        _pos_body,
        in_specs=[
            pl.BlockSpec((w, d), lambda: (0, 0)),
            pl.BlockSpec((h, d), lambda: (0, 0)),
        ],
        out_specs=pl.BlockSpec(memory_space=pl.ANY),
        out_shape=jax.ShapeDtypeStruct((b, h * w, 2 * d), jnp.float32),
        scratch_shapes=[
            pltpu.VMEM((h * w, 2 * d), jnp.float32),
            pltpu.SemaphoreType.DMA((b,)),
        ],
    )(col, row)
